# Initial kernel scaffold; baseline (speedup 1.0000x reference)
import functools

import jax
import jax.numpy as jnp
from jax import lax
from jax.experimental import pallas as pl
from jax.experimental.pallas import tpu as pltpu
from jax.experimental.pallas import tpu_sc as plsc

_B, _T, _C, _H, _M, _K = 1, 2048, 1024, 16, 8192, 3
_DH = _C // _H
_BT = 256
_BM = 1024


def _qkv_body(x_ref, w_ref, b_ref, o_ref):
    o_ref[...] = (
        jnp.dot(x_ref[...], w_ref[...], preferred_element_type=jnp.float32)
        + b_ref[...][None, :]
    )


def _qkv(x, w, b):
    return pl.pallas_call(
        _qkv_body,
        grid=(_T // _BT, 3),
        in_specs=[
            pl.BlockSpec((_BT, _C), lambda i, j: (i, 0)),
            pl.BlockSpec((_C, _C), lambda i, j: (0, j)),
            pl.BlockSpec((_C,), lambda i, j: (j,)),
        ],
        out_specs=pl.BlockSpec((_BT, _C), lambda i, j: (i, j)),
        out_shape=jax.ShapeDtypeStruct((_T, 3 * _C), jnp.float32),
    )(x, w, b)


def _attn_body(q_ref, k_ref, v_ref, o_ref):
    i = pl.program_id(1)
    q = q_ref[...]
    s = lax.dot_general(q, k_ref[...], (((1,), (1,)), ((), ())),
                        preferred_element_type=jnp.float32)
    s = s * (1.0 / jnp.sqrt(jnp.float32(_DH)))
    rows = i * _BT + lax.broadcasted_iota(jnp.int32, (_BT, _T), 0)
    cols = lax.broadcasted_iota(jnp.int32, (_BT, _T), 1)
    s = jnp.where(cols <= rows, s, jnp.float32(-1e30))
    m = jnp.max(s, axis=1, keepdims=True)
    e = jnp.exp(s - m)
    den = jnp.sum(e, axis=1, keepdims=True)
    o_ref[...] = (
        jnp.dot(e, v_ref[...], preferred_element_type=jnp.float32) / den
    )


def _attn(q, k, v):
    return pl.pallas_call(
        _attn_body,
        grid=(_H, _T // _BT),
        in_specs=[
            pl.BlockSpec((_BT, _DH), lambda h, i: (i, h)),
            pl.BlockSpec((_T, _DH), lambda h, i: (0, h)),
            pl.BlockSpec((_T, _DH), lambda h, i: (0, h)),
        ],
        out_specs=pl.BlockSpec((_BT, _DH), lambda h, i: (i, h)),
        out_shape=jax.ShapeDtypeStruct((_T, _C), jnp.float32),
    )(q, k, v)


def _topk_body(q_ref, kb_ref, o_ref, s_ref):
    j = pl.program_id(1)
    kb = kb_ref[...]
    kn = jnp.sum(kb * kb, axis=1)
    qk = lax.dot_general(q_ref[...], kb, (((1,), (1,)), ((), ())),
                         preferred_element_type=jnp.float32)
    s_ref[j] = kn[None, :] - 2.0 * qk

    @pl.when(j == _M // _BM - 1)
    def _():
        sc = jnp.concatenate([s_ref[jj] for jj in range(_M // _BM)], axis=1)
        cols = lax.broadcasted_iota(jnp.int32, (_BT, _M), 1)
        outs = []
        for _t in range(_K):
            mn = jnp.min(sc, axis=1, keepdims=True)
            im = jnp.min(jnp.where(sc == mn, cols, _M), axis=1, keepdims=True)
            outs.append(im)
            sc = jnp.where(cols == im, jnp.float32(jnp.inf), sc)
        o_ref[...] = jnp.concatenate(outs, axis=1).astype(jnp.int32)


def _topk(q, mem_flat):
    return pl.pallas_call(
        _topk_body,
        grid=(_T // _BT, _M // _BM),
        in_specs=[
            pl.BlockSpec((_BT, _C), lambda i, j: (i, 0)),
            pl.BlockSpec((_BM, _C), lambda i, j: (j, 0)),
        ],
        out_specs=pl.BlockSpec((_BT, _K), lambda i, j: (i, 0)),
        out_shape=jax.ShapeDtypeStruct((_T, _K), jnp.int32),
        scratch_shapes=[pltpu.VMEM((_M // _BM, _BT, _BM), jnp.float32)],
    )(q, mem_flat)


def _gather_rows(table, idx):
    info = plsc.get_sparse_core_info()
    nw = info.num_cores * info.num_subcores
    nrows = idx.shape[0]
    bpw = nrows // nw
    chunk = 24
    width = table.shape[1]
    mesh = plsc.VectorSubcoreMesh(core_axis_name="c", subcore_axis_name="s")

    @functools.partial(
        pl.kernel,
        mesh=mesh,
        out_type=jax.ShapeDtypeStruct((nrows, width), jnp.float32),
        scratch_types=[
            pltpu.VMEM((chunk,), jnp.int32),
            pltpu.VMEM((chunk, width), jnp.float32),
            pltpu.SemaphoreType.DMA,
        ],
    )
    def gk(table_hbm, idx_hbm, out_hbm, idx_v, rows_v, sem):
        wid = lax.axis_index("s") * info.num_cores + lax.axis_index("c")
        base = wid * bpw

        def body(c, carry):
            off = base + c * chunk
            pltpu.sync_copy(idx_hbm.at[pl.ds(off, chunk)], idx_v)
            pltpu.async_copy(table_hbm.at[idx_v], rows_v, sem).wait()
            pltpu.sync_copy(rows_v, out_hbm.at[pl.ds(off, chunk)])
            return carry

        lax.fori_loop(0, bpw // chunk, body, 0)

    return gk(table, idx)


def _combine_body(q_ref, y_ref, kv_ref, g_ref, w_ref, b_ref, o_ref):
    f32 = jnp.float32
    ch = lax.broadcasted_iota(jnp.int32, (_C, _H), 0) // _DH
    hh = lax.broadcasted_iota(jnp.int32, (_C, _H), 1)
    seg = (ch == hh).astype(f32)
    q = q_ref[...]
    logits = []
    for kk in range(_K):
        mk = kv_ref[:, kk, :_C]
        logits.append(
            jnp.dot(q * mk, seg, preferred_element_type=f32) * 0.125
        )
    m = jnp.maximum(jnp.maximum(logits[0], logits[1]), logits[2])
    es = [jnp.exp(l - m) for l in logits]
    den = es[0] + es[1] + es[2]
    mem = jnp.zeros((_BT, _C), f32)
    for kk in range(_K):
        w_full = lax.dot_general(es[kk] / den, seg, (((1,), (1,)), ((), ())),
                                 preferred_element_type=f32)
        mem = mem + w_full * kv_ref[:, kk, _C:]
    g = g_ref[...][None, :]
    comb = mem * g + y_ref[...] * (1.0 - g)
    o_ref[...] = (
        jnp.dot(comb, w_ref[...], preferred_element_type=f32)
        + b_ref[...][None, :]
    )


def _combine(q, y, kv3, gfull, wp, bp):
    return pl.pallas_call(
        _combine_body,
        grid=(_T // _BT,),
        in_specs=[
            pl.BlockSpec((_BT, _C), lambda i: (i, 0)),
            pl.BlockSpec((_BT, _C), lambda i: (i, 0)),
            pl.BlockSpec((_BT, _K, 2 * _C), lambda i: (i, 0, 0)),
            pl.BlockSpec((_C,), lambda i: (0,)),
            pl.BlockSpec((_C, _C), lambda i: (0, 0)),
            pl.BlockSpec((_C,), lambda i: (0,)),
        ],
        out_specs=pl.BlockSpec((_BT, _C), lambda i: (i, 0)),
        out_shape=jax.ShapeDtypeStruct((_T, _C), jnp.float32),
    )(q, y, kv3, gfull, wp, bp)


def kernel(x, memory_db, W_attn, b_attn, W_proj, b_proj, gate_bias):
    x2 = x.reshape(_T, _C)
    mem_flat = memory_db.reshape(_M, 2 * _C)
    qkv = _qkv(x2, W_attn, b_attn)
    q = qkv[:, :_C]
    k = qkv[:, _C:2 * _C]
    v = qkv[:, 2 * _C:]
    y = _attn(q, k, v)
    idx = _topk(q, mem_flat)
    kvs = _gather_rows(mem_flat, idx.reshape(_T * _K))
    kv3 = kvs.reshape(_T, _K, 2 * _C)
    gfull = jnp.repeat(gate_bias.reshape(_H), _DH)
    out = _combine(q, y, kv3, gfull, W_proj, b_proj)
    return out.reshape(_B, _T, _C)


# speedup vs baseline: 1.7998x; 1.7998x over previous
import functools

import jax
import jax.numpy as jnp
from jax import lax
from jax.experimental import pallas as pl
from jax.experimental.pallas import tpu as pltpu
from jax.experimental.pallas import tpu_sc as plsc

_B, _T, _C, _H, _M, _K = 1, 2048, 1024, 16, 8192, 3
_DH = _C // _H
_BT = 256
_BM = 1024


def _qkv_body(x_ref, w_ref, b_ref, o_ref):
    o_ref[...] = (
        jnp.dot(x_ref[...], w_ref[...], preferred_element_type=jnp.float32)
        + b_ref[...][None, :]
    )


def _qkv(x, w, b):
    return pl.pallas_call(
        _qkv_body,
        grid=(_T // _BT, 3),
        in_specs=[
            pl.BlockSpec((_BT, _C), lambda i, j: (i, 0)),
            pl.BlockSpec((_C, _C), lambda i, j: (0, j)),
            pl.BlockSpec((_C,), lambda i, j: (j,)),
        ],
        out_specs=pl.BlockSpec((_BT, _C), lambda i, j: (i, j)),
        out_shape=jax.ShapeDtypeStruct((_T, 3 * _C), jnp.float32),
    )(x, w, b)


def _attn_body(q_ref, kt_ref, v_ref, o_ref):
    i = pl.program_id(1)
    q = q_ref[0]
    s = jnp.dot(q, kt_ref[0], preferred_element_type=jnp.float32)
    s = s * (1.0 / jnp.sqrt(jnp.float32(_DH)))
    rows = i * _BT + lax.broadcasted_iota(jnp.int32, (_BT, _T), 0)
    cols = lax.broadcasted_iota(jnp.int32, (_BT, _T), 1)
    s = jnp.where(cols <= rows, s, jnp.float32(-1e30))
    m = jnp.max(s, axis=1, keepdims=True)
    e = jnp.exp(s - m)
    den = jnp.sum(e, axis=1, keepdims=True)
    o_ref[0] = (
        jnp.dot(e, v_ref[0], preferred_element_type=jnp.float32) / den
    )


def _attn(qh, kth, vh):
    return pl.pallas_call(
        _attn_body,
        grid=(_H, _T // _BT),
        in_specs=[
            pl.BlockSpec((1, _BT, _DH), lambda h, i: (h, i, 0)),
            pl.BlockSpec((1, _DH, _T), lambda h, i: (h, 0, 0)),
            pl.BlockSpec((1, _T, _DH), lambda h, i: (h, 0, 0)),
        ],
        out_specs=pl.BlockSpec((1, _BT, _DH), lambda h, i: (h, i, 0)),
        out_shape=jax.ShapeDtypeStruct((_H, _T, _DH), jnp.float32),
    )(qh, kth, vh)


def _topk_body(q_ref, kb_ref, o_ref, rv_ref, ri_ref):
    f32, i32 = jnp.float32, jnp.int32
    j = pl.program_id(1)

    @pl.when(j == 0)
    def _():
        rv_ref[...] = jnp.full((_BT, 8), jnp.inf, f32)
        ri_ref[...] = jnp.full((_BT, 8), 2 ** 30, i32)

    kbt = kb_ref[...]
    kn = jnp.sum(kbt * kbt, axis=0, keepdims=True)
    qk = jnp.dot(q_ref[...], kbt, preferred_element_type=f32)
    s = kn - 2.0 * qk

    cols = j * _BM + lax.broadcasted_iota(i32, (_BT, _BM), 1)
    lv, li = [], []
    for t in range(_K):
        mn = jnp.min(s, axis=1, keepdims=True)
        im = jnp.min(jnp.where(s == mn, cols, 2 ** 30), axis=1, keepdims=True)
        lv.append(mn)
        li.append(im)
        if t < _K - 1:
            s = jnp.where(cols == im, jnp.float32(jnp.inf), s)

    rv = rv_ref[...]
    ri = ri_ref[...]
    padv = jnp.full((_BT, 1), jnp.inf, f32)
    padi = jnp.full((_BT, 1), 2 ** 30, i32)
    cv = jnp.concatenate(
        [rv[:, 0:1], rv[:, 1:2], rv[:, 2:3]] + lv + [padv, padv], axis=1)
    ci = jnp.concatenate(
        [ri[:, 0:1], ri[:, 1:2], ri[:, 2:3]] + li + [padi, padi], axis=1)
    pos = lax.broadcasted_iota(i32, (_BT, 8), 1)
    nv, ni = [], []
    for _t in range(_K):
        mn = jnp.min(cv, axis=1, keepdims=True)
        p = jnp.min(jnp.where(cv == mn, pos, 8), axis=1, keepdims=True)
        nv.append(mn)
        ni.append(jnp.min(jnp.where(pos == p, ci, 2 ** 30), axis=1,
                          keepdims=True))
        cv = jnp.where(pos == p, jnp.float32(jnp.inf), cv)
    rv_ref[...] = jnp.concatenate(nv + [padv] * 5, axis=1)
    ri_ref[...] = jnp.concatenate(ni + [padi] * 5, axis=1)

    @pl.when(j == _M // _BM - 1)
    def _():
        o_ref[...] = jnp.concatenate(ni, axis=1)


def _topk(q, keys_t):
    return pl.pallas_call(
        _topk_body,
        grid=(_T // _BT, _M // _BM),
        in_specs=[
            pl.BlockSpec((_BT, _C), lambda i, j: (i, 0)),
            pl.BlockSpec((_C, _BM), lambda i, j: (0, j)),
        ],
        out_specs=pl.BlockSpec((_BT, _K), lambda i, j: (i, 0)),
        out_shape=jax.ShapeDtypeStruct((_T, _K), jnp.int32),
        scratch_shapes=[pltpu.VMEM((_BT, 8), jnp.float32),
                        pltpu.VMEM((_BT, 8), jnp.int32)],
    )(q, keys_t)


def _gather_rows(table, idx):
    info = plsc.get_sparse_core_info()
    nw = info.num_cores * info.num_subcores
    nrows = idx.shape[0]
    bpw = nrows // nw
    chunk = 24
    width = table.shape[1]
    mesh = plsc.VectorSubcoreMesh(core_axis_name="c", subcore_axis_name="s")

    @functools.partial(
        pl.kernel,
        mesh=mesh,
        out_type=jax.ShapeDtypeStruct((nrows, width), jnp.float32),
        scratch_types=[
            pltpu.VMEM((chunk,), jnp.int32),
            pltpu.VMEM((chunk, width), jnp.float32),
            pltpu.SemaphoreType.DMA,
        ],
    )
    def gk(table_hbm, idx_hbm, out_hbm, idx_v, rows_v, sem):
        wid = lax.axis_index("s") * info.num_cores + lax.axis_index("c")
        base = wid * bpw

        def body(c, carry):
            off = base + c * chunk
            pltpu.sync_copy(idx_hbm.at[pl.ds(off, chunk)], idx_v)
            pltpu.async_copy(table_hbm.at[idx_v], rows_v, sem).wait()
            pltpu.sync_copy(rows_v, out_hbm.at[pl.ds(off, chunk)])
            return carry

        lax.fori_loop(0, bpw // chunk, body, 0)

    return gk(table, idx)


def _combine_body(q_ref, y_ref, kv_ref, g_ref, w_ref, b_ref, o_ref):
    f32 = jnp.float32
    ch = lax.broadcasted_iota(jnp.int32, (_C, _H), 0) // _DH
    hh = lax.broadcasted_iota(jnp.int32, (_C, _H), 1)
    seg = (ch == hh).astype(f32)
    ch2 = lax.broadcasted_iota(jnp.int32, (_H, _C), 1) // _DH
    hh2 = lax.broadcasted_iota(jnp.int32, (_H, _C), 0)
    seg_t = (ch2 == hh2).astype(f32)
    q = q_ref[...]
    logits = []
    for kk in range(_K):
        mk = kv_ref[:, kk, :_C]
        logits.append(
            jnp.dot(q * mk, seg, preferred_element_type=f32) * 0.125
        )
    m = jnp.maximum(jnp.maximum(logits[0], logits[1]), logits[2])
    es = [jnp.exp(l - m) for l in logits]
    den = es[0] + es[1] + es[2]
    mem = jnp.zeros((_BT, _C), f32)
    for kk in range(_K):
        w_full = jnp.dot(es[kk] / den, seg_t,
                         preferred_element_type=f32)
        mem = mem + w_full * kv_ref[:, kk, _C:]
    g = g_ref[...][None, :]
    comb = mem * g + y_ref[...] * (1.0 - g)
    o_ref[...] = (
        jnp.dot(comb, w_ref[...], preferred_element_type=f32)
        + b_ref[...][None, :]
    )


def _combine(q, y, kv3, gfull, wp, bp):
    return pl.pallas_call(
        _combine_body,
        grid=(_T // _BT,),
        in_specs=[
            pl.BlockSpec((_BT, _C), lambda i: (i, 0)),
            pl.BlockSpec((_BT, _C), lambda i: (i, 0)),
            pl.BlockSpec((_BT, _K, 2 * _C), lambda i: (i, 0, 0)),
            pl.BlockSpec((_C,), lambda i: (0,)),
            pl.BlockSpec((_C, _C), lambda i: (0, 0)),
            pl.BlockSpec((_C,), lambda i: (0,)),
        ],
        out_specs=pl.BlockSpec((_BT, _C), lambda i: (i, 0)),
        out_shape=jax.ShapeDtypeStruct((_T, _C), jnp.float32),
    )(q, y, kv3, gfull, wp, bp)


def kernel(x, memory_db, W_attn, b_attn, W_proj, b_proj, gate_bias):
    x2 = x.reshape(_T, _C)
    mem_flat = memory_db.reshape(_M, 2 * _C)
    qkv = _qkv(x2, W_attn, b_attn)
    q = qkv[:, :_C]
    k = qkv[:, _C:2 * _C]
    v = qkv[:, 2 * _C:]
    qh = q.reshape(_T, _H, _DH).transpose(1, 0, 2)
    kth = k.reshape(_T, _H, _DH).transpose(1, 2, 0)
    vh = v.reshape(_T, _H, _DH).transpose(1, 0, 2)
    yh = _attn(qh, kth, vh)
    y = yh.transpose(1, 0, 2).reshape(_T, _C)
    keys_t = memory_db[:, 0, :].T
    idx = _topk(q, keys_t)
    kvs = _gather_rows(mem_flat, idx.reshape(_T * _K))
    kv3 = kvs.reshape(_T, _K, 2 * _C)
    gfull = jnp.repeat(gate_bias.reshape(_H), _DH)
    out = _combine(q, y, kv3, gfull, W_proj, b_proj)
    return out.reshape(_B, _T, _C)


# speedup vs baseline: 2.0280x; 1.1268x over previous
import functools

import jax
import jax.numpy as jnp
from jax import lax
from jax.experimental import pallas as pl
from jax.experimental.pallas import tpu as pltpu
from jax.experimental.pallas import tpu_sc as plsc

_B, _T, _C, _H, _M, _K = 1, 2048, 1024, 16, 8192, 3
_DH = _C // _H
_BT = 256
_BM = 1024


def _qkv_body(x_ref, w_ref, b_ref, o_ref):
    o_ref[...] = (
        jnp.dot(x_ref[...], w_ref[...], preferred_element_type=jnp.float32)
        + b_ref[...][None, :]
    )


def _qkv(x, w, b):
    return pl.pallas_call(
        _qkv_body,
        grid=(_T // _BT, 3),
        in_specs=[
            pl.BlockSpec((_BT, _C), lambda i, j: (i, 0)),
            pl.BlockSpec((_C, _C), lambda i, j: (0, j)),
            pl.BlockSpec((_C,), lambda i, j: (j,)),
        ],
        out_specs=pl.BlockSpec((_BT, _C), lambda i, j: (i, j)),
        out_shape=jax.ShapeDtypeStruct((_T, 3 * _C), jnp.float32),
    )(x, w, b)


def _attn_body(q_ref, kt_ref, v_ref, o_ref):
    i = pl.program_id(1)
    q = q_ref[0]
    s = jnp.dot(q, kt_ref[0], preferred_element_type=jnp.float32)
    s = s * (1.0 / jnp.sqrt(jnp.float32(_DH)))
    rows = i * _BT + lax.broadcasted_iota(jnp.int32, (_BT, _T), 0)
    cols = lax.broadcasted_iota(jnp.int32, (_BT, _T), 1)
    s = jnp.where(cols <= rows, s, jnp.float32(-1e30))
    m = jnp.max(s, axis=1, keepdims=True)
    e = jnp.exp(s - m)
    den = jnp.sum(e, axis=1, keepdims=True)
    o_ref[0] = (
        jnp.dot(e, v_ref[0], preferred_element_type=jnp.float32) / den
    )


def _attn(qh, kth, vh):
    return pl.pallas_call(
        _attn_body,
        grid=(_H, _T // _BT),
        in_specs=[
            pl.BlockSpec((1, _BT, _DH), lambda h, i: (h, i, 0)),
            pl.BlockSpec((1, _DH, _T), lambda h, i: (h, 0, 0)),
            pl.BlockSpec((1, _T, _DH), lambda h, i: (h, 0, 0)),
        ],
        out_specs=pl.BlockSpec((1, _BT, _DH), lambda h, i: (h, i, 0)),
        out_shape=jax.ShapeDtypeStruct((_H, _T, _DH), jnp.float32),
    )(qh, kth, vh)


def _topk_body(q_ref, kb_ref, o_ref, rv_ref, ri_ref):
    f32, i32 = jnp.float32, jnp.int32
    j = pl.program_id(1)

    @pl.when(j == 0)
    def _():
        rv_ref[...] = jnp.full((_BT, 8), jnp.inf, f32)
        ri_ref[...] = jnp.full((_BT, 8), 2 ** 30, i32)

    kbt = kb_ref[...]
    kn = jnp.sum(kbt * kbt, axis=0, keepdims=True)
    qk = jnp.dot(q_ref[...], kbt, preferred_element_type=f32)
    s = kn - 2.0 * qk

    cols = j * _BM + lax.broadcasted_iota(i32, (_BT, _BM), 1)
    lv, li = [], []
    for t in range(_K):
        mn = jnp.min(s, axis=1, keepdims=True)
        im = jnp.min(jnp.where(s == mn, cols, 2 ** 30), axis=1, keepdims=True)
        lv.append(mn)
        li.append(im)
        if t < _K - 1:
            s = jnp.where(cols == im, jnp.float32(jnp.inf), s)

    rv = rv_ref[...]
    ri = ri_ref[...]
    padv = jnp.full((_BT, 1), jnp.inf, f32)
    padi = jnp.full((_BT, 1), 2 ** 30, i32)
    cv = jnp.concatenate(
        [rv[:, 0:1], rv[:, 1:2], rv[:, 2:3]] + lv + [padv, padv], axis=1)
    ci = jnp.concatenate(
        [ri[:, 0:1], ri[:, 1:2], ri[:, 2:3]] + li + [padi, padi], axis=1)
    pos = lax.broadcasted_iota(i32, (_BT, 8), 1)
    nv, ni = [], []
    for _t in range(_K):
        mn = jnp.min(cv, axis=1, keepdims=True)
        p = jnp.min(jnp.where(cv == mn, pos, 8), axis=1, keepdims=True)
        nv.append(mn)
        ni.append(jnp.min(jnp.where(pos == p, ci, 2 ** 30), axis=1,
                          keepdims=True))
        cv = jnp.where(pos == p, jnp.float32(jnp.inf), cv)
    rv_ref[...] = jnp.concatenate(nv + [padv] * 5, axis=1)
    ri_ref[...] = jnp.concatenate(ni + [padi] * 5, axis=1)

    @pl.when(j == _M // _BM - 1)
    def _():
        o_ref[...] = jnp.concatenate(ni, axis=1)


def _topk(q, keys_t):
    return pl.pallas_call(
        _topk_body,
        grid=(_T // _BT, _M // _BM),
        in_specs=[
            pl.BlockSpec((_BT, _C), lambda i, j: (i, 0)),
            pl.BlockSpec((_C, _BM), lambda i, j: (0, j)),
        ],
        out_specs=pl.BlockSpec((_BT, _K), lambda i, j: (i, 0)),
        out_shape=jax.ShapeDtypeStruct((_T, _K), jnp.int32),
        scratch_shapes=[pltpu.VMEM((_BT, 8), jnp.float32),
                        pltpu.VMEM((_BT, 8), jnp.int32)],
    )(q, keys_t)


def _gather_rows(table, idx):
    info = plsc.get_sparse_core_info()
    nw = info.num_cores * info.num_subcores
    nrows = idx.shape[0]
    bpw = nrows // nw
    chunk = 24
    width = table.shape[1]
    mesh = plsc.VectorSubcoreMesh(core_axis_name="c", subcore_axis_name="s")

    @functools.partial(
        pl.kernel,
        mesh=mesh,
        out_type=jax.ShapeDtypeStruct((nrows, width), jnp.float32),
        scratch_types=[
            pltpu.VMEM((chunk,), jnp.int32),
            pltpu.VMEM((chunk, width), jnp.float32),
            pltpu.SemaphoreType.DMA,
        ],
    )
    def gk(table_hbm, idx_hbm, out_hbm, idx_v, rows_v, sem):
        wid = lax.axis_index("s") * info.num_cores + lax.axis_index("c")
        base = wid * bpw

        def body(c, carry):
            off = base + c * chunk
            pltpu.sync_copy(idx_hbm.at[pl.ds(off, chunk)], idx_v)
            pltpu.async_copy(table_hbm.at[idx_v], rows_v, sem).wait()
            pltpu.sync_copy(rows_v, out_hbm.at[pl.ds(off, chunk)])
            return carry

        lax.fori_loop(0, bpw // chunk, body, 0)

    return gk(table, idx)


def _combine_body(q_ref, y_ref, kv_ref, g_ref, w_ref, b_ref, o_ref):
    f32 = jnp.float32
    ch = lax.broadcasted_iota(jnp.int32, (_C, _H), 0) // _DH
    hh = lax.broadcasted_iota(jnp.int32, (_C, _H), 1)
    seg = (ch == hh).astype(f32)
    ch2 = lax.broadcasted_iota(jnp.int32, (_H, _C), 1) // _DH
    hh2 = lax.broadcasted_iota(jnp.int32, (_H, _C), 0)
    seg_t = (ch2 == hh2).astype(f32)
    q = q_ref[...]
    logits = []
    for kk in range(_K):
        mk = kv_ref[kk, :, :_C]
        logits.append(
            jnp.dot(q * mk, seg, preferred_element_type=f32) * 0.125
        )
    m = jnp.maximum(jnp.maximum(logits[0], logits[1]), logits[2])
    es = [jnp.exp(l - m) for l in logits]
    den = es[0] + es[1] + es[2]
    mem = jnp.zeros((_BT, _C), f32)
    for kk in range(_K):
        w_full = jnp.dot(es[kk] / den, seg_t,
                         preferred_element_type=f32)
        mem = mem + w_full * kv_ref[kk, :, _C:]
    g = g_ref[...][None, :]
    comb = mem * g + y_ref[...] * (1.0 - g)
    o_ref[...] = (
        jnp.dot(comb, w_ref[...], preferred_element_type=f32)
        + b_ref[...][None, :]
    )


def _combine(q, y, kv3, gfull, wp, bp):
    return pl.pallas_call(
        _combine_body,
        grid=(_T // _BT,),
        in_specs=[
            pl.BlockSpec((_BT, _C), lambda i: (i, 0)),
            pl.BlockSpec((_BT, _C), lambda i: (i, 0)),
            pl.BlockSpec((_K, _BT, 2 * _C), lambda i: (0, i, 0)),
            pl.BlockSpec((_C,), lambda i: (0,)),
            pl.BlockSpec((_C, _C), lambda i: (0, 0)),
            pl.BlockSpec((_C,), lambda i: (0,)),
        ],
        out_specs=pl.BlockSpec((_BT, _C), lambda i: (i, 0)),
        out_shape=jax.ShapeDtypeStruct((_T, _C), jnp.float32),
    )(q, y, kv3, gfull, wp, bp)


def kernel(x, memory_db, W_attn, b_attn, W_proj, b_proj, gate_bias):
    x2 = x.reshape(_T, _C)
    mem_flat = memory_db.reshape(_M, 2 * _C)
    qkv = _qkv(x2, W_attn, b_attn)
    q = qkv[:, :_C]
    k = qkv[:, _C:2 * _C]
    v = qkv[:, 2 * _C:]
    keys_t = memory_db[:, 0, :].T
    idx = _topk(q, keys_t)
    kvs = _gather_rows(mem_flat, idx.T.reshape(_K * _T))
    kv3 = kvs.reshape(_K, _T, 2 * _C)
    qh = q.reshape(_T, _H, _DH).transpose(1, 0, 2)
    kth = k.reshape(_T, _H, _DH).transpose(1, 2, 0)
    vh = v.reshape(_T, _H, _DH).transpose(1, 0, 2)
    yh = _attn(qh, kth, vh)
    y = yh.transpose(1, 0, 2).reshape(_T, _C)
    gfull = jnp.repeat(gate_bias.reshape(_H), _DH)
    out = _combine(q, y, kv3, gfull, W_proj, b_proj)
    return out.reshape(_B, _T, _C)
